# baseline (speedup 1.0000x reference)
import jax
import jax.numpy as jnp
from jax.experimental import pallas as pl
from jax.experimental.pallas import tpu as pltpu

N_MAP = 10000
N_AGT = 512
D_MAP = 128
D_CTX = 128
H = 6
D_H = H * D_CTX
DIST_TH = 0.06
_SCALE = D_CTX ** (-0.5)
_TILE = 2000


def _gn(x, g, b, eps=1e-5):
    m = jnp.mean(x, axis=-1, keepdims=True)
    ms = jnp.mean(x * x, axis=-1, keepdims=True)
    v = ms - m * m
    return (x - m) * jax.lax.rsqrt(v + eps) * g + b


def _relu(x):
    return jnp.maximum(x, 0.0)


def _dot(a, b):
    return jax.lax.dot_general(a, b, (((1,), (0,)), ((), ())),
                               preferred_element_type=jnp.float32)


def _dot_t(a, b):
    return jax.lax.dot_general(a, b, (((1,), (1,)), ((), ())),
                               preferred_element_type=jnp.float32)


_N_W = 20


def _layer_weights(p):
    r = lambda a: a.reshape(1, -1)
    hs = 0.5 * _SCALE
    wd = jnp.concatenate([0.5 * p['Wd'][:, 0], 0.5 * p['bd'],
                          jnp.zeros((1,), jnp.float32)]).reshape(1, 4)
    return [p['Wq'], r(hs * p['gq']), r(hs * p['bq']),
            p['Wk'], r(p['gk']), r(p['bk']),
            p['Wv'], r(p['gv']), r(p['bv']),
            wd,
            p['Wo1'], r(p['go1']), r(p['bo1']),
            p['Wo2'], p['Wagt'], r(p['gn_g']), r(p['gn_b']),
            p['Wlin'], r(p['glin']), r(p['blin'])]


def _body(feat_ref, mc_ref, ag_ref, act_ref, *rest):
    w0 = rest[:_N_W]
    w1 = rest[_N_W:2 * _N_W]
    out_ref = rest[2 * _N_W]
    kv_scratch = rest[2 * _N_W + 1:]

    @pl.when(pl.program_id(0) == 0)
    def _():
        a = ag_ref[...]
        for wl, ks, vs in ((w0, kv_scratch[0], kv_scratch[1]),
                           (w1, kv_scratch[2], kv_scratch[3])):
            ks[...] = _relu(_gn(_dot(a, wl[3][...]), wl[4][...], wl[5][...]))
            vs[...] = _relu(_gn(_dot(a, wl[6][...]), wl[7][...], wl[8][...]))

    mx = mc_ref[:, 0:1]
    my = mc_ref[:, 1:2]
    ax = act_ref[0:1, :]
    ay = act_ref[1:2, :]
    dx = mx - ax
    dy = my - ay
    in_range = dx * dx + dy * dy <= DIST_TH * DIST_TH

    x = feat_ref[...]
    for wl, ks, vs in ((w0, kv_scratch[0], kv_scratch[1]),
                       (w1, kv_scratch[2], kv_scratch[3])):
        res = x
        q = _relu(_gn(_dot(x, wl[0][...]), wl[1][...], wl[2][...]))
        wd = wl[9]
        bias = jnp.where(in_range, dx * wd[0, 0] + dy * wd[0, 1] + wd[0, 2],
                         -1e30)
        k = ks[...]
        v = vs[...]
        csum = _dot(jnp.ones((1, N_AGT), jnp.float32), v)
        outs = []
        for h in range(H):
            sl = slice(h * D_CTX, (h + 1) * D_CTX)
            t = jnp.tanh(_dot_t(q[:, sl], k[:, sl]) + bias)
            outs.append(_dot(t, v[:, sl]))
        o = jnp.concatenate(outs, axis=1) + csum
        o = _dot(_relu(_gn(_dot(o, wl[10][...]), wl[11][...], wl[12][...])),
                 wl[13][...])
        xx = _dot(x, wl[14][...]) + o
        xx = _relu(_gn(xx, wl[15][...], wl[16][...]))
        xx = _gn(_dot(xx, wl[17][...]), wl[18][...], wl[19][...])
        x = _relu(xx + res)
    out_ref[...] = x


def kernel(feat, map_ids, map_ctrs, agents, agent_ctrs, params):
    del map_ids
    act = agent_ctrs.T
    weights = _layer_weights(params['att0']) + _layer_weights(params['att1'])

    grid = (N_MAP // _TILE,)
    full = lambda a: pl.BlockSpec(a.shape, lambda i: (0,) * a.ndim)
    in_specs = [
        pl.BlockSpec((_TILE, D_MAP), lambda i: (i, 0)),
        pl.BlockSpec((_TILE, 2), lambda i: (i, 0)),
        full(agents),
        full(act),
    ] + [full(w) for w in weights]

    return pl.pallas_call(
        _body,
        grid=grid,
        in_specs=in_specs,
        out_specs=pl.BlockSpec((_TILE, D_MAP), lambda i: (i, 0)),
        out_shape=jax.ShapeDtypeStruct((N_MAP, D_MAP), jnp.float32),
        scratch_shapes=[pltpu.VMEM((N_AGT, D_H), jnp.float32)
                        for _ in range(4)],
    )(feat, map_ctrs, agents, act, *weights)


# speedup vs baseline: 1.0890x; 1.0890x over previous
import jax
import jax.numpy as jnp
from jax.experimental import pallas as pl
from jax.experimental.pallas import tpu as pltpu

N_MAP = 10000
N_AGT = 512
D_MAP = 128
D_CTX = 128
H = 6
D_H = H * D_CTX
DIST_TH = 0.06
_SCALE = D_CTX ** (-0.5)
_TILE = 2000


def _gn(x, g, b, eps=1e-5):
    m = jnp.mean(x, axis=-1, keepdims=True)
    ms = jnp.mean(x * x, axis=-1, keepdims=True)
    v = ms - m * m
    return (x - m) * jax.lax.rsqrt(v + eps) * g + b


def _relu(x):
    return jnp.maximum(x, 0.0)


def _dot(a, b):
    return jax.lax.dot_general(a, b, (((1,), (0,)), ((), ())),
                               preferred_element_type=jnp.float32)


def _dot_t(a, b):
    return jax.lax.dot_general(a, b, (((1,), (1,)), ((), ())),
                               preferred_element_type=jnp.float32)


_N_W = 20


def _layer_weights(p):
    r = lambda a: a.reshape(1, -1)
    hs = 0.5 * _SCALE
    wd = jnp.concatenate([0.5 * p['Wd'][:, 0], 0.5 * p['bd'],
                          jnp.zeros((1,), jnp.float32)]).reshape(1, 4)
    return [p['Wq'], r(hs * p['gq']), r(hs * p['bq']),
            p['Wk'], r(p['gk']), r(p['bk']),
            p['Wv'], r(p['gv']), r(p['bv']),
            wd,
            p['Wo1'], r(p['go1']), r(p['bo1']),
            p['Wo2'], p['Wagt'], r(p['gn_g']), r(p['gn_b']),
            p['Wlin'], r(p['glin']), r(p['blin'])]


def _body(feat_ref, mc_ref, ag_ref, act_ref, *rest):
    w0 = rest[:_N_W]
    w1 = rest[_N_W:2 * _N_W]
    out_ref = rest[2 * _N_W]
    kv_scratch = rest[2 * _N_W + 1:]

    @pl.when(pl.program_id(0) == 0)
    def _():
        a = ag_ref[...]
        for wl, ks, vs in ((w0, kv_scratch[0], kv_scratch[1]),
                           (w1, kv_scratch[2], kv_scratch[3])):
            ks[...] = _relu(_gn(_dot(a, wl[3][...]), wl[4][...], wl[5][...]))
            vs[...] = _relu(_gn(_dot(a, wl[6][...]), wl[7][...], wl[8][...]))

    mx = mc_ref[:, 0:1]
    my = mc_ref[:, 1:2]
    ax = act_ref[0:1, :]
    ay = act_ref[1:2, :]
    dx = mx - ax
    dy = my - ay
    penalty = jnp.where(dx * dx + dy * dy <= DIST_TH * DIST_TH, 0.0, -1e30)

    x = feat_ref[...]
    for wl, ks, vs in ((w0, kv_scratch[0], kv_scratch[1]),
                       (w1, kv_scratch[2], kv_scratch[3])):
        res = x
        q = _relu(_gn(_dot(x, wl[0][...]), wl[1][...], wl[2][...]))
        wd = wl[9]
        u = mx * wd[0, 0] + my * wd[0, 1]
        cvec = wd[0, 2] - ax * wd[0, 0] - ay * wd[0, 1]
        bias = (penalty + u) + cvec
        k = ks[...]
        v = vs[...]
        outs = []
        for h in range(H):
            sl = slice(h * D_CTX, (h + 1) * D_CTX)
            gates2 = jnp.tanh(_dot_t(q[:, sl], k[:, sl]) + bias) + 1.0
            outs.append(_dot(gates2, v[:, sl]))
        o = jnp.concatenate(outs, axis=1)
        o = _dot(_relu(_gn(_dot(o, wl[10][...]), wl[11][...], wl[12][...])),
                 wl[13][...])
        xx = _dot(x, wl[14][...]) + o
        xx = _relu(_gn(xx, wl[15][...], wl[16][...]))
        xx = _gn(_dot(xx, wl[17][...]), wl[18][...], wl[19][...])
        x = _relu(xx + res)
    out_ref[...] = x


def kernel(feat, map_ids, map_ctrs, agents, agent_ctrs, params):
    del map_ids
    act = agent_ctrs.T
    weights = _layer_weights(params['att0']) + _layer_weights(params['att1'])

    grid = (N_MAP // _TILE,)
    full = lambda a: pl.BlockSpec(a.shape, lambda i: (0,) * a.ndim)
    in_specs = [
        pl.BlockSpec((_TILE, D_MAP), lambda i: (i, 0)),
        pl.BlockSpec((_TILE, 2), lambda i: (i, 0)),
        full(agents),
        full(act),
    ] + [full(w) for w in weights]

    return pl.pallas_call(
        _body,
        grid=grid,
        in_specs=in_specs,
        out_specs=pl.BlockSpec((_TILE, D_MAP), lambda i: (i, 0)),
        out_shape=jax.ShapeDtypeStruct((N_MAP, D_MAP), jnp.float32),
        scratch_shapes=[pltpu.VMEM((N_AGT, D_H), jnp.float32)
                        for _ in range(4)],
    )(feat, map_ctrs, agents, act, *weights)


# speedup vs baseline: 1.1073x; 1.0168x over previous
import jax
import jax.numpy as jnp
from jax.experimental import pallas as pl
from jax.experimental.pallas import tpu as pltpu

N_MAP = 10000
N_AGT = 512
D_MAP = 128
D_CTX = 128
H = 6
D_H = H * D_CTX
DIST_TH = 0.06
_SCALE = D_CTX ** (-0.5)
_TILE = 2000


def _gn(x, g, b, eps=1e-5):
    m = jnp.mean(x, axis=-1, keepdims=True)
    ms = jnp.mean(x * x, axis=-1, keepdims=True)
    v = ms - m * m
    return (x - m) * jax.lax.rsqrt(v + eps) * g + b


def _relu(x):
    return jnp.maximum(x, 0.0)


def _dot(a, b):
    return jax.lax.dot_general(a, b, (((1,), (0,)), ((), ())),
                               preferred_element_type=jnp.float32)


def _dot_t(a, b):
    return jax.lax.dot_general(a, b, (((1,), (1,)), ((), ())),
                               preferred_element_type=jnp.float32)


_N_W = 5


def _layer_weights(p):
    hs = 0.5 * _SCALE
    qkv = jnp.concatenate([p['Wq'], p['Wk'], p['Wv']], axis=1)
    small = jnp.concatenate([p['Wo2'], p['Wagt'], p['Wlin']], axis=1)
    v768 = jnp.stack([hs * p['gq'], hs * p['bq'], p['gk'], p['bk'],
                      p['gv'], p['bv']], axis=0)
    wd = jnp.concatenate([0.5 * p['Wd'][:, 0], 0.5 * p['bd'],
                          jnp.zeros((125,), jnp.float32)])
    v128 = jnp.stack([p['go1'], p['bo1'], p['gn_g'], p['gn_b'],
                      p['glin'], p['blin'], wd], axis=0)
    return [qkv, p['Wo1'], small, v768, v128]


def _body(feat_ref, mc_ref, ag_ref, act_ref, *rest):
    w0 = rest[:_N_W]
    w1 = rest[_N_W:2 * _N_W]
    out_ref = rest[2 * _N_W]
    kv_scratch = rest[2 * _N_W + 1:]

    @pl.when(pl.program_id(0) == 0)
    def _():
        a = ag_ref[...]
        for wl, ks, vs in ((w0, kv_scratch[0], kv_scratch[1]),
                           (w1, kv_scratch[2], kv_scratch[3])):
            qkv, v768 = wl[0], wl[3]
            ks[...] = _relu(_gn(_dot(a, qkv[:, D_H:2 * D_H]),
                                v768[2:3, :], v768[3:4, :]))
            vs[...] = _relu(_gn(_dot(a, qkv[:, 2 * D_H:3 * D_H]),
                                v768[4:5, :], v768[5:6, :]))

    mx = mc_ref[:, 0:1]
    my = mc_ref[:, 1:2]
    ax = act_ref[0:1, :]
    ay = act_ref[1:2, :]
    dx = mx - ax
    dy = my - ay
    penalty = jnp.where(dx * dx + dy * dy <= DIST_TH * DIST_TH, 0.0, -1e30)

    x = feat_ref[...]
    for wl, ks, vs in ((w0, kv_scratch[0], kv_scratch[1]),
                       (w1, kv_scratch[2], kv_scratch[3])):
        res = x
        qkv, wo1, small, v768, v128 = (wl[0], wl[1], wl[2], wl[3], wl[4])
        q = _relu(_gn(_dot(x, qkv[:, 0:D_H]), v768[0:1, :], v768[1:2, :]))
        wd0 = v128[6, 0]
        wd1 = v128[6, 1]
        wd2 = v128[6, 2]
        u = mx * wd0 + my * wd1
        cvec = wd2 - ax * wd0 - ay * wd1
        bias = (penalty + u) + cvec
        k = ks[...]
        v = vs[...]
        outs = []
        for h in range(H):
            sl = slice(h * D_CTX, (h + 1) * D_CTX)
            gates2 = jnp.tanh(_dot_t(q[:, sl], k[:, sl]) + bias) + 1.0
            outs.append(_dot(gates2, v[:, sl]))
        o = jnp.concatenate(outs, axis=1)
        o = _dot(_relu(_gn(_dot(o, wo1[...]), v128[0:1, :], v128[1:2, :])),
                 small[:, 0:D_MAP])
        xx = _dot(x, small[:, D_MAP:2 * D_MAP]) + o
        xx = _relu(_gn(xx, v128[2:3, :], v128[3:4, :]))
        xx = _gn(_dot(xx, small[:, 2 * D_MAP:3 * D_MAP]),
                 v128[4:5, :], v128[5:6, :])
        x = _relu(xx + res)
    out_ref[...] = x


def kernel(feat, map_ids, map_ctrs, agents, agent_ctrs, params):
    del map_ids
    act = agent_ctrs.T
    weights = _layer_weights(params['att0']) + _layer_weights(params['att1'])

    grid = (N_MAP // _TILE,)
    full = lambda a: pl.BlockSpec(a.shape, lambda i: (0,) * a.ndim)
    in_specs = [
        pl.BlockSpec((_TILE, D_MAP), lambda i: (i, 0)),
        pl.BlockSpec((_TILE, 2), lambda i: (i, 0)),
        full(agents),
        full(act),
    ] + [full(w) for w in weights]

    return pl.pallas_call(
        _body,
        grid=grid,
        in_specs=in_specs,
        out_specs=pl.BlockSpec((_TILE, D_MAP), lambda i: (i, 0)),
        out_shape=jax.ShapeDtypeStruct((N_MAP, D_MAP), jnp.float32),
        scratch_shapes=[pltpu.VMEM((N_AGT, D_H), jnp.float32)
                        for _ in range(4)],
    )(feat, map_ctrs, agents, act, *weights)
